# baseline (speedup 1.0000x reference)
import functools

import jax
import jax.numpy as jnp
from jax import lax
from jax.experimental import pallas as pl
from jax.experimental.pallas import tpu as pltpu
from jax.experimental.pallas import tpu_sc as plsc

D = 128
NC, NS = 2, 16
NW = NC * NS
N_PAD = 10240
CNT_ROWS = N_PAD // D
ROWS_PER_TILE = N_PAD // NS
CHUNK = 128
PAD_NODE = N_PAD - 1

def _mesh():
    return plsc.VectorSubcoreMesh(core_axis_name="c", subcore_axis_name="s",
                                  num_cores=NC, num_subcores=NS)


def _make_bincount(n_chunk):
    seg = N_PAD // NS
    n_full, rem = divmod(CHUNK, 16)

    @functools.partial(
        pl.kernel, mesh=_mesh(),
        out_type=[jax.ShapeDtypeStruct((N_PAD,), jnp.float32),
                  jax.ShapeDtypeStruct((N_PAD,), jnp.float32)],
        scratch_types=[
            pltpu.VMEM((n_chunk, CHUNK), jnp.int32),
            pltpu.VMEM((N_PAD,), jnp.float32),
            pltpu.VMEM((seg,), jnp.float32),
            pltpu.VMEM_SHARED((NS, N_PAD), jnp.float32),
        ],
        compiler_params=pltpu.CompilerParams(needs_layout_passes=False),
    )
    def bincount(ei_hbm, out0_hbm, out1_hbm, src_v, hist_v, tmp_v, part_sh):
        cid = lax.axis_index("c")
        sid = lax.axis_index("s")
        wid = sid * NC + cid
        zero16 = jnp.zeros((16,), jnp.float32)

        def zrow(r, carry):
            hist_v[pl.ds(r * 16, 16)] = zero16
            return carry
        lax.fori_loop(0, N_PAD // 16, zrow, 0)
        pltpu.sync_copy(ei_hbm.at[0, wid], src_v)

        ones = jnp.ones((16,), jnp.float32)

        def grp(i, carry):
            for g in range(n_full):
                plsc.addupdate_scatter(
                    hist_v, [src_v[i, pl.ds(g * 16, 16)]], ones)
            return carry
        lax.fori_loop(0, n_chunk, grp, 0)

        pltpu.sync_copy(hist_v, part_sh.at[sid])
        plsc.subcore_barrier()
        base = sid * seg
        pltpu.sync_copy(part_sh.at[0, pl.ds(base, seg)], tmp_v)
        for j in range(seg // 16):
            hist_v[pl.ds(j * 16, 16)] = tmp_v[pl.ds(j * 16, 16)]
        for t in range(1, NS):
            pltpu.sync_copy(part_sh.at[t, pl.ds(base, seg)], tmp_v)
            for j in range(seg // 16):
                s = pl.ds(j * 16, 16)
                hist_v[s] = hist_v[s] + tmp_v[s]
        @pl.when(cid == 0)
        def _():
            pltpu.sync_copy(hist_v.at[pl.ds(0, seg)],
                            out0_hbm.at[pl.ds(base, seg)])

        @pl.when(cid == 1)
        def _():
            pltpu.sync_copy(hist_v.at[pl.ds(0, seg)],
                            out1_hbm.at[pl.ds(base, seg)])

    return bincount


def _make_aggregate(n_chunk):
    @functools.partial(
        pl.kernel, mesh=_mesh(),
        out_type=jax.ShapeDtypeStruct((NC, N_PAD, D), jnp.float32),
        scratch_types=[
            pltpu.VMEM((n_chunk // 2, CHUNK), jnp.int32),
            pltpu.VMEM((n_chunk // 2, CHUNK), jnp.int32),
            pltpu.VMEM((CHUNK, D), jnp.float32),
            pltpu.VMEM((CHUNK, D), jnp.float32),
            pltpu.VMEM_SHARED((N_PAD, D), jnp.float32),
            pltpu.SemaphoreType.DMA,
            pltpu.SemaphoreType.DMA,
        ],
        compiler_params=pltpu.CompilerParams(needs_layout_passes=False),
    )
    def aggregate(y_hbm, ei_hbm, zeros_hbm, out_hbm,
                  src_v, dst_v, rows0, rows1, acc_sh, sem0, sem1):
        cid = lax.axis_index("c")
        sid = lax.axis_index("s")
        wid = sid * NC + cid
        half = n_chunk // 2
        pltpu.sync_copy(zeros_hbm,
                        acc_sh.at[pl.ds(sid * ROWS_PER_TILE, ROWS_PER_TILE)])
        plsc.subcore_barrier()

        last = half - 1
        for h in range(2):
            pltpu.sync_copy(ei_hbm.at[0, wid, pl.ds(h * half, half)], src_v)
            pltpu.sync_copy(ei_hbm.at[1, wid, pl.ds(h * half, half)], dst_v)
            pltpu.async_copy(y_hbm.at[src_v.at[0]], rows0, sem0)

            def pair(i, carry):
                j = i * 2
                pltpu.async_copy(y_hbm.at[src_v.at[j + 1]], rows1, sem1)
                pltpu.make_async_copy(y_hbm.at[src_v.at[j]], rows0, sem0).wait()
                pltpu.sync_copy(rows0, acc_sh.at[dst_v.at[j]], add=True)
                nxt = jnp.minimum(j + 2, last)
                pltpu.async_copy(y_hbm.at[src_v.at[nxt]], rows0, sem0)
                pltpu.make_async_copy(y_hbm.at[src_v.at[j + 1]], rows1,
                                      sem1).wait()
                pltpu.sync_copy(rows1, acc_sh.at[dst_v.at[j + 1]], add=True)
                return carry
            lax.fori_loop(0, half // 2, pair, 0)
            pltpu.make_async_copy(y_hbm.at[src_v.at[last]], rows0, sem0).wait()

        plsc.subcore_barrier()
        pltpu.sync_copy(acc_sh.at[pl.ds(sid * ROWS_PER_TILE, ROWS_PER_TILE)],
                        out_hbm.at[cid, pl.ds(sid * ROWS_PER_TILE, ROWS_PER_TILE)])

    return aggregate


_BLK = 1024


def _matmul_body(x_ref, w_ref, b_ref, re_ref, z_ref, s_ref):
    xl = lax.dot_general(x_ref[...], w_ref[...], (((1,), (1,)), ((), ())),
                         preferred_element_type=jnp.float32) + b_ref[...]
    z_ref[...] = jnp.maximum(xl, 0.0)
    s_ref[...] = jnp.maximum(xl + re_ref[...], 0.0)


def _scale_body(cnt0_ref, cnt1_ref, z_ref, y_ref):
    deg = cnt0_ref[...] + cnt1_ref[...] + 1.0
    y_ref[...] = lax.rsqrt(deg) * z_ref[...]


def _combine_body(cnt0_ref, cnt1_ref, acc_ref, s_ref, out_ref):
    deg = cnt0_ref[...] + cnt1_ref[...] + 1.0
    out_ref[...] = (lax.rsqrt(deg) * (acc_ref[0] + acc_ref[1])
                    + s_ref[...] / deg)


def kernel(x, edge_index, W, b, root_emb):
    n, d = x.shape
    e = edge_index.shape[1]
    assert d == D and n <= N_PAD
    e_pad = -(-e // (NW * CHUNK * 2)) * NW * CHUNK * 2
    n_chunk = e_pad // (NW * CHUNK)
    ei = edge_index.astype(jnp.int32)
    if e_pad != e:
        ei = jnp.concatenate(
            [ei, jnp.full((2, e_pad - e), PAD_NODE, jnp.int32)], axis=1)
    ei4 = ei.reshape(2, NW, n_chunk, CHUNK)
    b2 = b.reshape(1, D)

    grid = (N_PAD // _BLK,)
    col_spec = pl.BlockSpec((_BLK, 1), lambda i: (i, 0))
    row_spec = pl.BlockSpec((_BLK, D), lambda i: (i, 0))
    full_spec = pl.BlockSpec((D, D), lambda i: (0, 0))
    vec_spec = pl.BlockSpec((1, D), lambda i: (0, 0))
    acc_spec = pl.BlockSpec((2, _BLK, D), lambda i: (0, i, 0))

    c0, c1 = _make_bincount(n_chunk)(ei4)
    cnt0 = c0.reshape(N_PAD, 1)
    cnt1 = c1.reshape(N_PAD, 1)

    z, s = pl.pallas_call(
        _matmul_body,
        grid=grid,
        in_specs=[row_spec, full_spec, vec_spec, vec_spec],
        out_specs=[row_spec, row_spec],
        out_shape=[
            jax.ShapeDtypeStruct((n, D), jnp.float32),
            jax.ShapeDtypeStruct((n, D), jnp.float32),
        ],
    )(x, W, b2, root_emb)

    y = pl.pallas_call(
        _scale_body,
        grid=grid,
        in_specs=[col_spec, col_spec, row_spec],
        out_specs=row_spec,
        out_shape=jax.ShapeDtypeStruct((N_PAD, D), jnp.float32),
    )(cnt0, cnt1, z)

    zeros = jnp.zeros((ROWS_PER_TILE, D), jnp.float32)
    acc = _make_aggregate(n_chunk)(y, ei4, zeros)

    out = pl.pallas_call(
        _combine_body,
        grid=grid,
        in_specs=[col_spec, col_spec, acc_spec, row_spec],
        out_specs=row_spec,
        out_shape=jax.ShapeDtypeStruct((n, D), jnp.float32),
    )(cnt0, cnt1, acc, s)
    return out


# speedup vs baseline: 2.6851x; 2.6851x over previous
import functools

import jax
import jax.numpy as jnp
from jax import lax
from jax.experimental import pallas as pl
from jax.experimental.pallas import tpu as pltpu
from jax.experimental.pallas import tpu_sc as plsc

D = 128
NC, NS = 2, 16
NW = NC * NS
N_PAD = 10240
CNT_ROWS = N_PAD // D
ROWS_PER_TILE = N_PAD // NS
CHUNK = 128
PAD_NODE = N_PAD - 1

def _mesh():
    return plsc.VectorSubcoreMesh(core_axis_name="c", subcore_axis_name="s",
                                  num_cores=NC, num_subcores=NS)


def _make_bincount(n_chunk):
    seg = N_PAD // NS
    n_full, rem = divmod(CHUNK, 16)

    @functools.partial(
        pl.kernel, mesh=_mesh(),
        out_type=[jax.ShapeDtypeStruct((N_PAD,), jnp.float32),
                  jax.ShapeDtypeStruct((N_PAD,), jnp.float32)],
        scratch_types=[
            pltpu.VMEM((n_chunk, CHUNK), jnp.int32),
            pltpu.VMEM((N_PAD,), jnp.float32),
            pltpu.VMEM((seg,), jnp.float32),
            pltpu.VMEM_SHARED((NS, N_PAD), jnp.float32),
        ],
        compiler_params=pltpu.CompilerParams(needs_layout_passes=False),
    )
    def bincount(ei_hbm, out0_hbm, out1_hbm, src_v, hist_v, tmp_v, part_sh):
        cid = lax.axis_index("c")
        sid = lax.axis_index("s")
        wid = sid * NC + cid
        zero16 = jnp.zeros((16,), jnp.float32)

        def zrow(r, carry):
            hist_v[pl.ds(r * 16, 16)] = zero16
            return carry
        lax.fori_loop(0, N_PAD // 16, zrow, 0)
        pltpu.sync_copy(ei_hbm.at[0, wid], src_v)

        ones = jnp.ones((16,), jnp.float32)

        def grp(i, carry):
            for g in range(n_full):
                plsc.addupdate_scatter(
                    hist_v, [src_v[i, pl.ds(g * 16, 16)]], ones)
            return carry
        lax.fori_loop(0, n_chunk, grp, 0)

        pltpu.sync_copy(hist_v, part_sh.at[sid])
        plsc.subcore_barrier()
        base = sid * seg
        pltpu.sync_copy(part_sh.at[0, pl.ds(base, seg)], tmp_v)
        for j in range(seg // 16):
            hist_v[pl.ds(j * 16, 16)] = tmp_v[pl.ds(j * 16, 16)]
        for t in range(1, NS):
            pltpu.sync_copy(part_sh.at[t, pl.ds(base, seg)], tmp_v)
            for j in range(seg // 16):
                s = pl.ds(j * 16, 16)
                hist_v[s] = hist_v[s] + tmp_v[s]
        @pl.when(cid == 0)
        def _():
            pltpu.sync_copy(hist_v.at[pl.ds(0, seg)],
                            out0_hbm.at[pl.ds(base, seg)])

        @pl.when(cid == 1)
        def _():
            pltpu.sync_copy(hist_v.at[pl.ds(0, seg)],
                            out1_hbm.at[pl.ds(base, seg)])

    return bincount


def _make_aggregate(n_chunk):
    @functools.partial(
        pl.kernel, mesh=_mesh(),
        out_type=jax.ShapeDtypeStruct((NC, N_PAD, D), jnp.float32),
        scratch_types=[
            pltpu.VMEM((n_chunk // 2, CHUNK), jnp.int32),
            pltpu.VMEM((n_chunk // 2, CHUNK), jnp.int32),
            pltpu.VMEM((CHUNK, D), jnp.float32),
            pltpu.VMEM((CHUNK, D), jnp.float32),
            pltpu.VMEM_SHARED((N_PAD, D), jnp.float32),
            pltpu.SemaphoreType.DMA,
            pltpu.SemaphoreType.DMA,
        ],
        compiler_params=pltpu.CompilerParams(needs_layout_passes=False),
    )
    def aggregate(y_hbm, ei_hbm, zeros_hbm, out_hbm,
                  src_v, dst_v, rows0, rows1, acc_sh, sem0, sem1):
        cid = lax.axis_index("c")
        sid = lax.axis_index("s")
        wid = sid * NC + cid
        half = n_chunk // 2
        pltpu.sync_copy(zeros_hbm,
                        acc_sh.at[pl.ds(sid * ROWS_PER_TILE, ROWS_PER_TILE)])
        plsc.subcore_barrier()

        last = half - 1
        for h in range(2):
            pltpu.sync_copy(ei_hbm.at[0, wid, pl.ds(h * half, half)], src_v)
            pltpu.sync_copy(ei_hbm.at[1, wid, pl.ds(h * half, half)], dst_v)
            pltpu.async_copy(y_hbm.at[src_v.at[0]], rows0, sem0)

            def pair(i, carry):
                j = i * 2
                pltpu.async_copy(y_hbm.at[src_v.at[j + 1]], rows1, sem1)
                pltpu.make_async_copy(y_hbm.at[src_v.at[j]], rows0, sem0).wait()
                pltpu.sync_copy(rows0, acc_sh.at[dst_v.at[j]], add=True)
                nxt = jnp.minimum(j + 2, last)
                pltpu.async_copy(y_hbm.at[src_v.at[nxt]], rows0, sem0)
                pltpu.make_async_copy(y_hbm.at[src_v.at[j + 1]], rows1,
                                      sem1).wait()
                pltpu.sync_copy(rows1, acc_sh.at[dst_v.at[j + 1]], add=True)
                return carry
            lax.fori_loop(0, half // 2, pair, 0)
            pltpu.make_async_copy(y_hbm.at[src_v.at[last]], rows0, sem0).wait()

        plsc.subcore_barrier()
        pltpu.sync_copy(acc_sh.at[pl.ds(sid * ROWS_PER_TILE, ROWS_PER_TILE)],
                        out_hbm.at[cid, pl.ds(sid * ROWS_PER_TILE, ROWS_PER_TILE)])

    return aggregate


_BLK = 1024


def _matmul_body(x_ref, w_ref, b_ref, re_ref, z_ref, s_ref):
    xl = lax.dot_general(x_ref[...], w_ref[...], (((1,), (1,)), ((), ())),
                         preferred_element_type=jnp.float32) + b_ref[...]
    z_ref[...] = jnp.maximum(xl, 0.0)
    s_ref[...] = jnp.maximum(xl + re_ref[...], 0.0)


def _scale_body(cnt0_ref, cnt1_ref, z_ref, y_ref):
    deg = cnt0_ref[...] + cnt1_ref[...] + 1.0
    y_ref[...] = lax.rsqrt(deg) * z_ref[...]


def _combine_body(cnt0_ref, cnt1_ref, acc_ref, s_ref, out_ref):
    deg = cnt0_ref[...] + cnt1_ref[...] + 1.0
    out_ref[...] = (lax.rsqrt(deg) * (acc_ref[0] + acc_ref[1])
                    + s_ref[...] / deg)


def kernel(x, edge_index, W, b, root_emb):
    n, d = x.shape
    e = edge_index.shape[1]
    assert d == D and n <= N_PAD
    e_pad = -(-e // (NW * CHUNK * 2)) * NW * CHUNK * 2
    n_chunk = e_pad // (NW * CHUNK)
    ei = edge_index.astype(jnp.int32)
    if e_pad != e:
        pad_rows = N_PAD - n
        pad_idx = n + jax.lax.iota(jnp.int32, e_pad - e) % pad_rows
        ei = jnp.concatenate(
            [ei, jnp.broadcast_to(pad_idx, (2, e_pad - e))], axis=1)
    ei4 = ei.reshape(2, NW, n_chunk, CHUNK)
    b2 = b.reshape(1, D)

    grid = (N_PAD // _BLK,)
    col_spec = pl.BlockSpec((_BLK, 1), lambda i: (i, 0))
    row_spec = pl.BlockSpec((_BLK, D), lambda i: (i, 0))
    full_spec = pl.BlockSpec((D, D), lambda i: (0, 0))
    vec_spec = pl.BlockSpec((1, D), lambda i: (0, 0))
    acc_spec = pl.BlockSpec((2, _BLK, D), lambda i: (0, i, 0))

    c0, c1 = _make_bincount(n_chunk)(ei4)
    cnt0 = c0.reshape(N_PAD, 1)
    cnt1 = c1.reshape(N_PAD, 1)

    z, s = pl.pallas_call(
        _matmul_body,
        grid=grid,
        in_specs=[row_spec, full_spec, vec_spec, vec_spec],
        out_specs=[row_spec, row_spec],
        out_shape=[
            jax.ShapeDtypeStruct((n, D), jnp.float32),
            jax.ShapeDtypeStruct((n, D), jnp.float32),
        ],
    )(x, W, b2, root_emb)

    y = pl.pallas_call(
        _scale_body,
        grid=grid,
        in_specs=[col_spec, col_spec, row_spec],
        out_specs=row_spec,
        out_shape=jax.ShapeDtypeStruct((N_PAD, D), jnp.float32),
    )(cnt0, cnt1, z)

    zeros = jnp.zeros((ROWS_PER_TILE, D), jnp.float32)
    acc = _make_aggregate(n_chunk)(y, ei4, zeros)

    out = pl.pallas_call(
        _combine_body,
        grid=grid,
        in_specs=[col_spec, col_spec, acc_spec, row_spec],
        out_specs=row_spec,
        out_shape=jax.ShapeDtypeStruct((n, D), jnp.float32),
    )(cnt0, cnt1, acc, s)
    return out


# speedup vs baseline: 2.9412x; 1.0954x over previous
import functools

import jax
import jax.numpy as jnp
from jax import lax
from jax.experimental import pallas as pl
from jax.experimental.pallas import tpu as pltpu
from jax.experimental.pallas import tpu_sc as plsc

D = 128
NC, NS = 2, 16
NW = NC * NS
N_PAD = 10240
CNT_ROWS = N_PAD // D
ROWS_PER_TILE = N_PAD // NS
CHUNK = 125

def _mesh():
    return plsc.VectorSubcoreMesh(core_axis_name="c", subcore_axis_name="s",
                                  num_cores=NC, num_subcores=NS)


def _make_bincount(e):
    seg = N_PAD // NS
    n_tiles = e // 128
    q, r = divmod(n_tiles, NW)
    big = (q + 1) * 128
    small = q * 128

    @functools.partial(
        pl.kernel, mesh=_mesh(),
        out_type=[jax.ShapeDtypeStruct((N_PAD,), jnp.float32),
                  jax.ShapeDtypeStruct((N_PAD,), jnp.float32)],
        scratch_types=[
            pltpu.VMEM((big,), jnp.int32),
            pltpu.VMEM((N_PAD,), jnp.float32),
            pltpu.VMEM((seg,), jnp.float32),
            pltpu.VMEM_SHARED((NS, N_PAD), jnp.float32),
        ],
        compiler_params=pltpu.CompilerParams(needs_layout_passes=False),
    )
    def bincount(ei_hbm, out0_hbm, out1_hbm, src_v, hist_v, tmp_v, part_sh):
        cid = lax.axis_index("c")
        sid = lax.axis_index("s")
        wid = sid * NC + cid
        zero16 = jnp.zeros((16,), jnp.float32)

        def zrow(i, carry):
            hist_v[pl.ds(i * 16, 16)] = zero16
            return carry
        lax.fori_loop(0, N_PAD // 16, zrow, 0)

        start = (wid * q + jnp.minimum(wid, r)) * 128
        mine = jnp.where(wid < r, big, small)

        @pl.when(wid < r)
        def _():
            pltpu.sync_copy(ei_hbm.at[0, pl.ds(start, big)], src_v)

        @pl.when(wid >= r)
        def _():
            pltpu.sync_copy(ei_hbm.at[0, pl.ds(start, small)],
                            src_v.at[pl.ds(0, small)])

        ones = jnp.ones((16,), jnp.float32)

        def grp(i, carry):
            plsc.addupdate_scatter(hist_v, [src_v[pl.ds(i * 16, 16)]], ones)
            return carry
        lax.fori_loop(0, mine // 16, grp, 0)

        pltpu.sync_copy(hist_v, part_sh.at[sid])
        plsc.subcore_barrier()
        base = sid * seg
        pltpu.sync_copy(part_sh.at[0, pl.ds(base, seg)], tmp_v)
        for j in range(seg // 16):
            hist_v[pl.ds(j * 16, 16)] = tmp_v[pl.ds(j * 16, 16)]
        for t in range(1, NS):
            pltpu.sync_copy(part_sh.at[t, pl.ds(base, seg)], tmp_v)
            for j in range(seg // 16):
                s = pl.ds(j * 16, 16)
                hist_v[s] = hist_v[s] + tmp_v[s]
        @pl.when(cid == 0)
        def _():
            pltpu.sync_copy(hist_v.at[pl.ds(0, seg)],
                            out0_hbm.at[pl.ds(base, seg)])

        @pl.when(cid == 1)
        def _():
            pltpu.sync_copy(hist_v.at[pl.ds(0, seg)],
                            out1_hbm.at[pl.ds(base, seg)])

    return bincount


def _make_aggregate(n_chunk):
    @functools.partial(
        pl.kernel, mesh=_mesh(),
        out_type=jax.ShapeDtypeStruct((NC, N_PAD, D), jnp.float32),
        scratch_types=[
            pltpu.VMEM((n_chunk // 2, CHUNK), jnp.int32),
            pltpu.VMEM((n_chunk // 2, CHUNK), jnp.int32),
            pltpu.VMEM((CHUNK, D), jnp.float32),
            pltpu.VMEM((CHUNK, D), jnp.float32),
            pltpu.VMEM_SHARED((N_PAD, D), jnp.float32),
            pltpu.SemaphoreType.DMA,
            pltpu.SemaphoreType.DMA,
        ],
        compiler_params=pltpu.CompilerParams(needs_layout_passes=False),
    )
    def aggregate(y_hbm, ei_hbm, zeros_hbm, out_hbm,
                  src_v, dst_v, rows0, rows1, acc_sh, sem0, sem1):
        cid = lax.axis_index("c")
        sid = lax.axis_index("s")
        wid = sid * NC + cid
        half = n_chunk // 2
        pltpu.sync_copy(zeros_hbm,
                        acc_sh.at[pl.ds(sid * ROWS_PER_TILE, ROWS_PER_TILE)])
        plsc.subcore_barrier()

        last = half - 1
        for h in range(2):
            pltpu.sync_copy(ei_hbm.at[0, wid, pl.ds(h * half, half)], src_v)
            pltpu.sync_copy(ei_hbm.at[1, wid, pl.ds(h * half, half)], dst_v)
            pltpu.async_copy(y_hbm.at[src_v.at[0]], rows0, sem0)

            def pair(i, carry):
                j = i * 2
                pltpu.async_copy(y_hbm.at[src_v.at[j + 1]], rows1, sem1)
                pltpu.make_async_copy(y_hbm.at[src_v.at[j]], rows0, sem0).wait()
                pltpu.sync_copy(rows0, acc_sh.at[dst_v.at[j]], add=True)
                nxt = jnp.minimum(j + 2, last)
                pltpu.async_copy(y_hbm.at[src_v.at[nxt]], rows0, sem0)
                pltpu.make_async_copy(y_hbm.at[src_v.at[j + 1]], rows1,
                                      sem1).wait()
                pltpu.sync_copy(rows1, acc_sh.at[dst_v.at[j + 1]], add=True)
                return carry
            lax.fori_loop(0, half // 2, pair, 0)
            pltpu.make_async_copy(y_hbm.at[src_v.at[last]], rows0, sem0).wait()

        plsc.subcore_barrier()
        pltpu.sync_copy(acc_sh.at[pl.ds(sid * ROWS_PER_TILE, ROWS_PER_TILE)],
                        out_hbm.at[cid, pl.ds(sid * ROWS_PER_TILE, ROWS_PER_TILE)])

    return aggregate


_BLK = 1024


def _matmul_body(x_ref, w_ref, b_ref, re_ref, z_ref, s_ref):
    xl = lax.dot_general(x_ref[...], w_ref[...], (((1,), (1,)), ((), ())),
                         preferred_element_type=jnp.float32) + b_ref[...]
    z_ref[...] = jnp.maximum(xl, 0.0)
    s_ref[...] = jnp.maximum(xl + re_ref[...], 0.0)


def _deg_col(cnt0_ref, cnt1_ref):
    deg8 = cnt0_ref[...] + cnt1_ref[...] + 1.0
    sel = (lax.broadcasted_iota(jnp.int32, (_BLK, 8), 0) // 128
           == lax.broadcasted_iota(jnp.int32, (_BLK, 8), 1)
           ).astype(jnp.float32)
    rep = lax.dot_general(sel, deg8, (((1,), (0,)), ((), ())),
                          preferred_element_type=jnp.float32)
    keep = (lax.broadcasted_iota(jnp.int32, (_BLK, D), 0) % 128
            == lax.broadcasted_iota(jnp.int32, (_BLK, D), 1))
    return jnp.sum(jnp.where(keep, rep, 0.0), axis=1, keepdims=True)


def _scale_body(cnt0_ref, cnt1_ref, z_ref, y_ref):
    deg = _deg_col(cnt0_ref, cnt1_ref)
    y_ref[...] = lax.rsqrt(deg) * z_ref[...]


def _combine_body(cnt0_ref, cnt1_ref, acc_ref, s_ref, out_ref):
    deg = _deg_col(cnt0_ref, cnt1_ref)
    out_ref[...] = (lax.rsqrt(deg) * (acc_ref[0] + acc_ref[1])
                    + s_ref[...] / deg)


def kernel(x, edge_index, W, b, root_emb):
    n, d = x.shape
    e = edge_index.shape[1]
    assert d == D and n <= N_PAD
    e_per_w = e // NW
    assert e % 128 == 0 and e % NW == 0 and e_per_w % CHUNK == 0
    n_chunk = e_per_w // CHUNK

    ei = edge_index.astype(jnp.int32)
    ei4 = ei.reshape(2, NW, n_chunk, CHUNK)
    b2 = b.reshape(1, D)

    grid = (N_PAD // _BLK,)
    col_spec = pl.BlockSpec((8, D), lambda i: (i, 0))
    row_spec = pl.BlockSpec((_BLK, D), lambda i: (i, 0))
    full_spec = pl.BlockSpec((D, D), lambda i: (0, 0))
    vec_spec = pl.BlockSpec((1, D), lambda i: (0, 0))
    acc_spec = pl.BlockSpec((2, _BLK, D), lambda i: (0, i, 0))

    c0, c1 = _make_bincount(e)(ei)
    cnt0 = c0.reshape(N_PAD // D, D)
    cnt1 = c1.reshape(N_PAD // D, D)

    z, s = pl.pallas_call(
        _matmul_body,
        grid=grid,
        in_specs=[row_spec, full_spec, vec_spec, vec_spec],
        out_specs=[row_spec, row_spec],
        out_shape=[
            jax.ShapeDtypeStruct((n, D), jnp.float32),
            jax.ShapeDtypeStruct((n, D), jnp.float32),
        ],
    )(x, W, b2, root_emb)

    y = pl.pallas_call(
        _scale_body,
        grid=grid,
        in_specs=[col_spec, col_spec, row_spec],
        out_specs=row_spec,
        out_shape=jax.ShapeDtypeStruct((n, D), jnp.float32),
    )(cnt0, cnt1, z)

    zeros = jnp.zeros((ROWS_PER_TILE, D), jnp.float32)
    acc = _make_aggregate(n_chunk)(y, ei4, zeros)

    out = pl.pallas_call(
        _combine_body,
        grid=grid,
        in_specs=[col_spec, col_spec, acc_spec, row_spec],
        out_specs=row_spec,
        out_shape=jax.ShapeDtypeStruct((n, D), jnp.float32),
    )(cnt0, cnt1, acc, s)
    return out


# speedup vs baseline: 2.9662x; 1.0085x over previous
import functools

import jax
import jax.numpy as jnp
from jax import lax
from jax.experimental import pallas as pl
from jax.experimental.pallas import tpu as pltpu
from jax.experimental.pallas import tpu_sc as plsc

D = 128
NC, NS = 2, 16
NW = NC * NS
N_PAD = 10240
CNT_ROWS = N_PAD // D
ROWS_PER_TILE = N_PAD // NS
CHUNK = 125

def _mesh():
    return plsc.VectorSubcoreMesh(core_axis_name="c", subcore_axis_name="s",
                                  num_cores=NC, num_subcores=NS)


def _make_bincount(e):
    seg = N_PAD // NS
    n_tiles = e // 128
    q, r = divmod(n_tiles, NW)
    big = (q + 1) * 128
    small = q * 128

    @functools.partial(
        pl.kernel, mesh=_mesh(),
        out_type=[jax.ShapeDtypeStruct((N_PAD,), jnp.float32),
                  jax.ShapeDtypeStruct((N_PAD,), jnp.float32)],
        scratch_types=[
            pltpu.VMEM((big,), jnp.int32),
            pltpu.VMEM((N_PAD,), jnp.float32),
            pltpu.VMEM((NS, seg), jnp.float32),
            pltpu.VMEM_SHARED((NS, N_PAD), jnp.float32),
        ],
        compiler_params=pltpu.CompilerParams(needs_layout_passes=False),
    )
    def bincount(ei_hbm, out0_hbm, out1_hbm, src_v, hist_v, tmp_v, part_sh):
        cid = lax.axis_index("c")
        sid = lax.axis_index("s")
        wid = sid * NC + cid
        zero16 = jnp.zeros((16,), jnp.float32)

        def zrow(i, carry):
            hist_v[pl.ds(i * 16, 16)] = zero16
            return carry
        lax.fori_loop(0, N_PAD // 16, zrow, 0)

        start = (wid * q + jnp.minimum(wid, r)) * 128
        mine = jnp.where(wid < r, big, small)

        @pl.when(wid < r)
        def _():
            pltpu.sync_copy(ei_hbm.at[0, pl.ds(start, big)], src_v)

        @pl.when(wid >= r)
        def _():
            pltpu.sync_copy(ei_hbm.at[0, pl.ds(start, small)],
                            src_v.at[pl.ds(0, small)])

        ones = jnp.ones((16,), jnp.float32)

        def grp(i, carry):
            plsc.addupdate_scatter(hist_v, [src_v[pl.ds(i * 16, 16)]], ones)
            return carry
        lax.fori_loop(0, mine // 16, grp, 0)

        pltpu.sync_copy(hist_v, part_sh.at[sid])
        plsc.subcore_barrier()
        base = sid * seg
        pltpu.sync_copy(part_sh.at[:, pl.ds(base, seg)], tmp_v)
        for j in range(seg // 16):
            s = pl.ds(j * 16, 16)
            acc16 = tmp_v[0, s]
            for t in range(1, NS):
                acc16 = acc16 + tmp_v[t, s]
            hist_v[s] = acc16
        @pl.when(cid == 0)
        def _():
            pltpu.sync_copy(hist_v.at[pl.ds(0, seg)],
                            out0_hbm.at[pl.ds(base, seg)])

        @pl.when(cid == 1)
        def _():
            pltpu.sync_copy(hist_v.at[pl.ds(0, seg)],
                            out1_hbm.at[pl.ds(base, seg)])

    return bincount


def _make_aggregate(n_chunk):
    @functools.partial(
        pl.kernel, mesh=_mesh(),
        out_type=jax.ShapeDtypeStruct((NC, N_PAD, D), jnp.float32),
        scratch_types=[
            pltpu.VMEM((n_chunk // 2, CHUNK), jnp.int32),
            pltpu.VMEM((n_chunk // 2, CHUNK), jnp.int32),
            pltpu.VMEM((CHUNK, D), jnp.float32),
            pltpu.VMEM((CHUNK, D), jnp.float32),
            pltpu.VMEM_SHARED((N_PAD, D), jnp.float32),
            pltpu.SemaphoreType.DMA,
            pltpu.SemaphoreType.DMA,
        ],
        compiler_params=pltpu.CompilerParams(needs_layout_passes=False),
    )
    def aggregate(y_hbm, ei_hbm, zeros_hbm, out_hbm,
                  src_v, dst_v, rows0, rows1, acc_sh, sem0, sem1):
        cid = lax.axis_index("c")
        sid = lax.axis_index("s")
        wid = sid * NC + cid
        half = n_chunk // 2
        zcp = pltpu.async_copy(
            zeros_hbm, acc_sh.at[pl.ds(sid * ROWS_PER_TILE, ROWS_PER_TILE)],
            sem1)

        last = half - 1
        for h in range(2):
            pltpu.sync_copy(ei_hbm.at[0, wid, pl.ds(h * half, half)], src_v)
            pltpu.sync_copy(ei_hbm.at[1, wid, pl.ds(h * half, half)], dst_v)
            if h == 0:
                zcp.wait()
                plsc.subcore_barrier()
            pltpu.async_copy(y_hbm.at[src_v.at[0]], rows0, sem0)

            def pair(i, carry):
                j = i * 2
                pltpu.async_copy(y_hbm.at[src_v.at[j + 1]], rows1, sem1)
                pltpu.make_async_copy(y_hbm.at[src_v.at[j]], rows0, sem0).wait()
                pltpu.sync_copy(rows0, acc_sh.at[dst_v.at[j]], add=True)
                nxt = jnp.minimum(j + 2, last)
                pltpu.async_copy(y_hbm.at[src_v.at[nxt]], rows0, sem0)
                pltpu.make_async_copy(y_hbm.at[src_v.at[j + 1]], rows1,
                                      sem1).wait()
                pltpu.sync_copy(rows1, acc_sh.at[dst_v.at[j + 1]], add=True)
                return carry
            lax.fori_loop(0, half // 2, pair, 0)
            pltpu.make_async_copy(y_hbm.at[src_v.at[last]], rows0, sem0).wait()

        plsc.subcore_barrier()
        pltpu.sync_copy(acc_sh.at[pl.ds(sid * ROWS_PER_TILE, ROWS_PER_TILE)],
                        out_hbm.at[cid, pl.ds(sid * ROWS_PER_TILE, ROWS_PER_TILE)])

    return aggregate


_BLK = 1024


def _matmul_body(x_ref, w_ref, b_ref, re_ref, z_ref, s_ref):
    xl = lax.dot_general(x_ref[...], w_ref[...], (((1,), (1,)), ((), ())),
                         preferred_element_type=jnp.float32) + b_ref[...]
    z_ref[...] = jnp.maximum(xl, 0.0)
    s_ref[...] = jnp.maximum(xl + re_ref[...], 0.0)


def _deg_col(cnt0_ref, cnt1_ref):
    deg8 = cnt0_ref[...] + cnt1_ref[...] + 1.0
    sel = (lax.broadcasted_iota(jnp.int32, (_BLK, 8), 0) // 128
           == lax.broadcasted_iota(jnp.int32, (_BLK, 8), 1)
           ).astype(jnp.float32)
    rep = lax.dot_general(sel, deg8, (((1,), (0,)), ((), ())),
                          preferred_element_type=jnp.float32)
    keep = (lax.broadcasted_iota(jnp.int32, (_BLK, D), 0) % 128
            == lax.broadcasted_iota(jnp.int32, (_BLK, D), 1))
    return jnp.sum(jnp.where(keep, rep, 0.0), axis=1, keepdims=True)


def _scale_body(cnt0_ref, cnt1_ref, z_ref, y_ref):
    deg = _deg_col(cnt0_ref, cnt1_ref)
    y_ref[...] = lax.rsqrt(deg) * z_ref[...]


def _combine_body(cnt0_ref, cnt1_ref, acc_ref, s_ref, out_ref):
    deg = _deg_col(cnt0_ref, cnt1_ref)
    out_ref[...] = (lax.rsqrt(deg) * (acc_ref[0] + acc_ref[1])
                    + s_ref[...] / deg)


def kernel(x, edge_index, W, b, root_emb):
    n, d = x.shape
    e = edge_index.shape[1]
    assert d == D and n <= N_PAD
    e_per_w = e // NW
    assert e % 128 == 0 and e % NW == 0 and e_per_w % CHUNK == 0
    n_chunk = e_per_w // CHUNK

    ei = edge_index.astype(jnp.int32)
    ei4 = ei.reshape(2, NW, n_chunk, CHUNK)
    b2 = b.reshape(1, D)

    grid = (N_PAD // _BLK,)
    col_spec = pl.BlockSpec((8, D), lambda i: (i, 0))
    row_spec = pl.BlockSpec((_BLK, D), lambda i: (i, 0))
    full_spec = pl.BlockSpec((D, D), lambda i: (0, 0))
    vec_spec = pl.BlockSpec((1, D), lambda i: (0, 0))
    acc_spec = pl.BlockSpec((2, _BLK, D), lambda i: (0, i, 0))

    c0, c1 = _make_bincount(e)(ei)
    cnt0 = c0.reshape(N_PAD // D, D)
    cnt1 = c1.reshape(N_PAD // D, D)

    z, s = pl.pallas_call(
        _matmul_body,
        grid=grid,
        in_specs=[row_spec, full_spec, vec_spec, vec_spec],
        out_specs=[row_spec, row_spec],
        out_shape=[
            jax.ShapeDtypeStruct((n, D), jnp.float32),
            jax.ShapeDtypeStruct((n, D), jnp.float32),
        ],
    )(x, W, b2, root_emb)

    y = pl.pallas_call(
        _scale_body,
        grid=grid,
        in_specs=[col_spec, col_spec, row_spec],
        out_specs=row_spec,
        out_shape=jax.ShapeDtypeStruct((n, D), jnp.float32),
    )(cnt0, cnt1, z)

    zeros = jnp.zeros((ROWS_PER_TILE, D), jnp.float32)
    acc = _make_aggregate(n_chunk)(y, ei4, zeros)

    out = pl.pallas_call(
        _combine_body,
        grid=grid,
        in_specs=[col_spec, col_spec, acc_spec, row_spec],
        out_specs=row_spec,
        out_shape=jax.ShapeDtypeStruct((n, D), jnp.float32),
    )(cnt0, cnt1, acc, s)
    return out


# speedup vs baseline: 2.9735x; 1.0025x over previous
import functools

import jax
import jax.numpy as jnp
from jax import lax
from jax.experimental import pallas as pl
from jax.experimental.pallas import tpu as pltpu
from jax.experimental.pallas import tpu_sc as plsc

D = 128
NC, NS = 2, 16
NW = NC * NS
N_PAD = 10240
ROWS_PER_TILE = N_PAD // NS
CHUNK = 125

def _mesh():
    return plsc.VectorSubcoreMesh(core_axis_name="c", subcore_axis_name="s",
                                  num_cores=NC, num_subcores=NS)


def _make_bincount(e):
    seg = N_PAD // NS
    n_tiles = e // 128
    q, r = divmod(n_tiles, NW)
    big = (q + 1) * 128
    small = q * 128

    @functools.partial(
        pl.kernel, mesh=_mesh(),
        out_type=[jax.ShapeDtypeStruct((N_PAD,), jnp.float32),
                  jax.ShapeDtypeStruct((N_PAD,), jnp.float32)],
        scratch_types=[
            pltpu.VMEM((big,), jnp.int32),
            pltpu.VMEM((N_PAD,), jnp.float32),
            pltpu.VMEM((NS, seg), jnp.float32),
            pltpu.VMEM_SHARED((NS, N_PAD), jnp.float32),
        ],
        compiler_params=pltpu.CompilerParams(needs_layout_passes=False),
    )
    def bincount(ei_hbm, out0_hbm, out1_hbm, src_v, hist_v, tmp_v, part_sh):
        cid = lax.axis_index("c")
        sid = lax.axis_index("s")
        wid = sid * NC + cid
        zero16 = jnp.zeros((16,), jnp.float32)

        def zrow(i, carry):
            hist_v[pl.ds(i * 16, 16)] = zero16
            return carry
        lax.fori_loop(0, N_PAD // 16, zrow, 0)

        start = (wid * q + jnp.minimum(wid, r)) * 128
        mine = jnp.where(wid < r, big, small)

        @pl.when(wid < r)
        def _():
            pltpu.sync_copy(ei_hbm.at[0, pl.ds(start, big)], src_v)

        @pl.when(wid >= r)
        def _():
            pltpu.sync_copy(ei_hbm.at[0, pl.ds(start, small)],
                            src_v.at[pl.ds(0, small)])

        ones = jnp.ones((16,), jnp.float32)

        def grp(i, carry):
            plsc.addupdate_scatter(hist_v, [src_v[pl.ds(i * 16, 16)]], ones)
            return carry
        lax.fori_loop(0, mine // 16, grp, 0)

        pltpu.sync_copy(hist_v, part_sh.at[sid])
        plsc.subcore_barrier()
        base = sid * seg
        pltpu.sync_copy(part_sh.at[:, pl.ds(base, seg)], tmp_v)
        for j in range(seg // 16):
            s = pl.ds(j * 16, 16)
            acc16 = tmp_v[0, s]
            for t in range(1, NS):
                acc16 = acc16 + tmp_v[t, s]
            hist_v[s] = acc16
        @pl.when(cid == 0)
        def _():
            pltpu.sync_copy(hist_v.at[pl.ds(0, seg)],
                            out0_hbm.at[pl.ds(base, seg)])

        @pl.when(cid == 1)
        def _():
            pltpu.sync_copy(hist_v.at[pl.ds(0, seg)],
                            out1_hbm.at[pl.ds(base, seg)])

    return bincount


def _make_aggregate(n_chunk):
    @functools.partial(
        pl.kernel, mesh=_mesh(),
        out_type=jax.ShapeDtypeStruct((NC, N_PAD, D), jnp.float32),
        scratch_types=[
            pltpu.VMEM((n_chunk // 2, CHUNK), jnp.int32),
            pltpu.VMEM((n_chunk // 2, CHUNK), jnp.int32),
            pltpu.VMEM((CHUNK, D), jnp.float32),
            pltpu.VMEM((CHUNK, D), jnp.float32),
            pltpu.VMEM_SHARED((N_PAD, D), jnp.float32),
            pltpu.SemaphoreType.DMA,
            pltpu.SemaphoreType.DMA,
        ],
        compiler_params=pltpu.CompilerParams(needs_layout_passes=False),
    )
    def aggregate(y_hbm, ei_hbm, zeros_hbm, out_hbm,
                  src_v, dst_v, rows0, rows1, acc_sh, sem0, sem1):
        cid = lax.axis_index("c")
        sid = lax.axis_index("s")
        wid = sid * NC + cid
        half = n_chunk // 2
        zcp = pltpu.async_copy(
            zeros_hbm, acc_sh.at[pl.ds(sid * ROWS_PER_TILE, ROWS_PER_TILE)],
            sem1)

        last = half - 1
        for h in range(2):
            pltpu.sync_copy(ei_hbm.at[0, wid, pl.ds(h * half, half)], src_v)
            pltpu.sync_copy(ei_hbm.at[1, wid, pl.ds(h * half, half)], dst_v)
            if h == 0:
                zcp.wait()
                plsc.subcore_barrier()
            pltpu.async_copy(y_hbm.at[src_v.at[0]], rows0, sem0)

            def pair(i, carry):
                j = i * 2
                pltpu.async_copy(y_hbm.at[src_v.at[j + 1]], rows1, sem1)
                pltpu.make_async_copy(y_hbm.at[src_v.at[j]], rows0, sem0).wait()
                pltpu.sync_copy(rows0, acc_sh.at[dst_v.at[j]], add=True)
                nxt = jnp.minimum(j + 2, last)
                pltpu.async_copy(y_hbm.at[src_v.at[nxt]], rows0, sem0)
                pltpu.make_async_copy(y_hbm.at[src_v.at[j + 1]], rows1,
                                      sem1).wait()
                pltpu.sync_copy(rows1, acc_sh.at[dst_v.at[j + 1]], add=True)
                return carry
            lax.fori_loop(0, half // 2, pair, 0)
            pltpu.make_async_copy(y_hbm.at[src_v.at[last]], rows0, sem0).wait()

        plsc.subcore_barrier()
        pltpu.sync_copy(acc_sh.at[pl.ds(sid * ROWS_PER_TILE, ROWS_PER_TILE)],
                        out_hbm.at[cid, pl.ds(sid * ROWS_PER_TILE, ROWS_PER_TILE)])

    return aggregate


_BLK = 1024


def _matmul_body(x_ref, w_ref, b_ref, re_ref, z_ref, s_ref):
    xl = lax.dot_general(x_ref[...], w_ref[...], (((1,), (1,)), ((), ())),
                         preferred_element_type=jnp.float32) + b_ref[...]
    z_ref[...] = jnp.maximum(xl, 0.0)
    s_ref[...] = jnp.maximum(xl + re_ref[...], 0.0)


def _deg_col(cnt0_ref, cnt1_ref):
    deg8 = cnt0_ref[...] + cnt1_ref[...] + 1.0
    sel = (lax.broadcasted_iota(jnp.int32, (_BLK, 8), 0) // 128
           == lax.broadcasted_iota(jnp.int32, (_BLK, 8), 1)
           ).astype(jnp.float32)
    rep = lax.dot_general(sel, deg8, (((1,), (0,)), ((), ())),
                          preferred_element_type=jnp.float32)
    keep = (lax.broadcasted_iota(jnp.int32, (_BLK, D), 0) % 128
            == lax.broadcasted_iota(jnp.int32, (_BLK, D), 1))
    return jnp.sum(jnp.where(keep, rep, 0.0), axis=1, keepdims=True)


def _scale_body(cnt0_ref, cnt1_ref, z_ref, y_ref):
    deg = _deg_col(cnt0_ref, cnt1_ref)
    y_ref[...] = lax.rsqrt(deg) * z_ref[...]


def _combine_body(cnt0_ref, cnt1_ref, acc_ref, s_ref, out_ref):
    deg = _deg_col(cnt0_ref, cnt1_ref)
    out_ref[...] = (lax.rsqrt(deg) * (acc_ref[0] + acc_ref[1])
                    + s_ref[...] / deg)


def kernel(x, edge_index, W, b, root_emb):
    n, d = x.shape
    e = edge_index.shape[1]
    assert d == D and n <= N_PAD
    e_per_w = e // NW
    assert e % 128 == 0 and e % NW == 0 and e_per_w % CHUNK == 0
    n_chunk = e_per_w // CHUNK

    ei = edge_index.astype(jnp.int32)
    ei4 = ei.reshape(2, NW, n_chunk, CHUNK)
    b2 = b.reshape(1, D)

    grid = (N_PAD // _BLK,)
    col_spec = pl.BlockSpec((8, D), lambda i: (i, 0))
    row_spec = pl.BlockSpec((_BLK, D), lambda i: (i, 0))
    full_spec = pl.BlockSpec((D, D), lambda i: (0, 0))
    vec_spec = pl.BlockSpec((1, D), lambda i: (0, 0))
    acc_spec = pl.BlockSpec((2, _BLK, D), lambda i: (0, i, 0))

    c0, c1 = _make_bincount(e)(ei)
    cnt0 = c0.reshape(N_PAD // D, D)
    cnt1 = c1.reshape(N_PAD // D, D)

    z, s = pl.pallas_call(
        _matmul_body,
        grid=grid,
        in_specs=[row_spec, full_spec, vec_spec, vec_spec],
        out_specs=[row_spec, row_spec],
        out_shape=[
            jax.ShapeDtypeStruct((n, D), jnp.float32),
            jax.ShapeDtypeStruct((n, D), jnp.float32),
        ],
    )(x, W, b2, root_emb)

    y = pl.pallas_call(
        _scale_body,
        grid=grid,
        in_specs=[col_spec, col_spec, row_spec],
        out_specs=row_spec,
        out_shape=jax.ShapeDtypeStruct((n, D), jnp.float32),
    )(cnt0, cnt1, z)

    zeros = jnp.zeros((ROWS_PER_TILE, D), jnp.float32)
    acc = _make_aggregate(n_chunk)(y, ei4, zeros)

    out = pl.pallas_call(
        _combine_body,
        grid=grid,
        in_specs=[col_spec, col_spec, acc_spec, row_spec],
        out_specs=row_spec,
        out_shape=jax.ShapeDtypeStruct((n, D), jnp.float32),
    )(cnt0, cnt1, acc, s)
    return out
